# Initial kernel scaffold; baseline (speedup 1.0000x reference)
import jax
import jax.numpy as jnp
from jax.experimental import pallas as pl


def kernel(node_attrs, node_feats, edge_attrs, edge_feats, edge_index, W_up, W1, W2, W3, W4, W_lin0, W_lin1):
    raise NotImplementedError("write your pallas kernel here")


# speedup vs baseline: 2.0236x; 2.0236x over previous
import functools

import jax
import jax.numpy as jnp
import numpy as np
from jax import lax
from jax.experimental import pallas as pl
from jax.experimental.pallas import tpu as pltpu
from jax.experimental.pallas import tpu_sc as plsc

N_NODES = 10000
N_EDGES = 320000
CH = 128
AVG_NEIGH = 32.0

_NSUB = 16
_EPT = N_EDGES // _NSUB
_B = 80
_NB = _EPT // _B
_ROWS = 624
_TAIL = N_NODES - _NSUB * _ROWS
_ZR = 104


def _linup_body(nf_ref, w_ref, o_ref):
    o_ref[...] = jnp.dot(nf_ref[...], w_ref[...],
                         preferred_element_type=jnp.float32)


def _linear_up(node_feats, w_up_s):
    bn = 2000
    return pl.pallas_call(
        _linup_body,
        grid=(N_NODES // bn,),
        in_specs=[
            pl.BlockSpec((bn, CH), lambda i: (i, 0)),
            pl.BlockSpec((CH, CH), lambda i: (0, 0)),
        ],
        out_specs=pl.BlockSpec((bn, CH), lambda i: (i, 0)),
        out_shape=jax.ShapeDtypeStruct((N_NODES, CH), jnp.float32),
    )(node_feats, w_up_s)


def _edgew_body(ef_ref, ea_ref, w1_ref, w2_ref, w3_ref, w4_ref, wv_ref):
    h = jax.nn.silu(jnp.dot(ef_ref[...], w1_ref[...],
                            preferred_element_type=jnp.float32))
    h = jax.nn.silu(jnp.dot(h, w2_ref[...],
                            preferred_element_type=jnp.float32))
    h = jax.nn.silu(jnp.dot(h, w3_ref[...],
                            preferred_element_type=jnp.float32))
    tpw = jnp.dot(h, w4_ref[...], preferred_element_type=jnp.float32)
    ea = ea_ref[...]
    w0 = tpw[:, :CH]
    w1t = tpw[:, CH:]
    wv_ref[0] = w0 * ea[:, 0:1]
    wv_ref[1] = w1t * ea[:, 1:2]
    wv_ref[2] = w1t * ea[:, 2:3]
    wv_ref[3] = w1t * ea[:, 3:4]


def _edge_weights(edge_feats, edge_attrs, w1s, w2s, w3s, w4s):
    be = 4000
    return pl.pallas_call(
        _edgew_body,
        grid=(N_EDGES // be,),
        in_specs=[
            pl.BlockSpec((be, 8), lambda i: (i, 0)),
            pl.BlockSpec((be, 4), lambda i: (i, 0)),
            pl.BlockSpec((8, 64), lambda i: (0, 0)),
            pl.BlockSpec((64, 64), lambda i: (0, 0)),
            pl.BlockSpec((64, 64), lambda i: (0, 0)),
            pl.BlockSpec((64, 2 * CH), lambda i: (0, 0)),
        ],
        out_specs=pl.BlockSpec((4, be, CH), lambda i: (0, i, 0)),
        out_shape=jax.ShapeDtypeStruct((4, N_EDGES, CH), jnp.float32),
    )(edge_feats, edge_attrs, w1s, w2s, w3s, w4s)


def _sc_body(snd_hbm, rcv_hbm, x_hbm, wv_hbm, out_hbm,
             snd_v, rcv_v, xs_v, wv_v, prod_v, zero_v, acc_sh, gsem):
    cid = lax.axis_index("c")
    sid = lax.axis_index("s")
    ebase = sid * _EPT

    def zrow(i, carry):
        for k in range(CH // 16):
            zero_v[i, pl.ds(k * 16, 16)] = jnp.zeros((16,), jnp.float32)
        return carry
    lax.fori_loop(0, _ZR, zrow, 0)

    for r in range(2):
        chunk = r * 2 + cid
        for z in range(_ROWS // _ZR):
            pltpu.sync_copy(zero_v,
                            acc_sh.at[pl.ds(sid * _ROWS + z * _ZR, _ZR)])

        @pl.when(sid == 0)
        def _zero_tail():
            pltpu.sync_copy(zero_v.at[pl.ds(0, _TAIL)],
                            acc_sh.at[pl.ds(_NSUB * _ROWS, _TAIL)])
        plsc.subcore_barrier()

        def batch(i, carry):
            eb = ebase + i * _B
            pltpu.sync_copy(snd_hbm.at[pl.ds(eb, _B)], snd_v)
            pltpu.sync_copy(rcv_hbm.at[pl.ds(eb, _B)], rcv_v)
            pltpu.async_copy(x_hbm.at[snd_v], xs_v, gsem).wait()
            pltpu.sync_copy(wv_hbm.at[pl.ds(chunk * N_EDGES + eb, _B)], wv_v)

            def edge(e, c2):
                for k in range(CH // 16):
                    a = xs_v[e, pl.ds(k * 16, 16)]
                    b = wv_v[e, pl.ds(k * 16, 16)]
                    prod_v[e, pl.ds(k * 16, 16)] = a * b
                return c2
            lax.fori_loop(0, _B, edge, 0)
            pltpu.sync_copy(prod_v, acc_sh.at[rcv_v], add=True)
            return carry
        lax.fori_loop(0, _NB, batch, 0)
        plsc.subcore_barrier()
        pltpu.sync_copy(acc_sh.at[pl.ds(sid * _ROWS, _ROWS)],
                        out_hbm.at[pl.ds(chunk * N_NODES + sid * _ROWS,
                                         _ROWS)])

        @pl.when(sid == 0)
        def _dump_tail():
            pltpu.sync_copy(
                acc_sh.at[pl.ds(_NSUB * _ROWS, _TAIL)],
                out_hbm.at[pl.ds(chunk * N_NODES + _NSUB * _ROWS, _TAIL)])


def _sc_message(snd, rcv, x, wv2d):
    mesh = plsc.VectorSubcoreMesh(core_axis_name="c", subcore_axis_name="s")
    k = functools.partial(
        pl.kernel,
        mesh=mesh,
        out_type=jax.ShapeDtypeStruct((4 * N_NODES, CH), jnp.float32),
        scratch_types=[
            pltpu.VMEM((_B,), jnp.int32),
            pltpu.VMEM((_B,), jnp.int32),
            pltpu.VMEM((_B, CH), jnp.float32),
            pltpu.VMEM((_B, CH), jnp.float32),
            pltpu.VMEM((_B, CH), jnp.float32),
            pltpu.VMEM((_ZR, CH), jnp.float32),
            pltpu.VMEM_SHARED((N_NODES, CH), jnp.float32),
            pltpu.SemaphoreType.DMA,
        ],
    )(_sc_body)
    return k(snd, rcv, x, wv2d)


def _outlin_body(m_ref, w_ref, o_ref):
    o_ref[0] = jnp.dot(m_ref[0], w_ref[0],
                       preferred_element_type=jnp.float32)


def _out_linear(msg, w_stack):
    bn = 2000
    return pl.pallas_call(
        _outlin_body,
        grid=(4, N_NODES // bn),
        in_specs=[
            pl.BlockSpec((1, bn, CH), lambda c, i: (c, i, 0)),
            pl.BlockSpec((1, CH, CH), lambda c, i: (c, 0, 0)),
        ],
        out_specs=pl.BlockSpec((1, bn, CH), lambda c, i: (c, i, 0)),
        out_shape=jax.ShapeDtypeStruct((4, N_NODES, CH), jnp.float32),
    )(msg, w_stack)


def kernel(node_attrs, node_feats, edge_attrs, edge_feats, edge_index,
           W_up, W1, W2, W3, W4, W_lin0, W_lin1):
    del node_attrs
    snd = edge_index[0]
    rcv = edge_index[1]
    w_up_s = W_up * np.float32(1.0 / np.sqrt(CH))
    w1s = W1 * np.float32(1.0 / np.sqrt(8.0))
    w2s = W2 * np.float32(1.0 / np.sqrt(64.0))
    w3s = W3 * np.float32(1.0 / np.sqrt(64.0))
    w4s = W4 * np.float32(1.0 / np.sqrt(64.0))
    out_scale = np.float32(1.0 / (np.sqrt(CH) * AVG_NEIGH))
    w_stack = jnp.stack([W_lin0, W_lin1, W_lin1, W_lin1], axis=0) * out_scale

    x = _linear_up(node_feats, w_up_s)
    wv = _edge_weights(edge_feats, edge_attrs, w1s, w2s, w3s, w4s)
    msg2d = _sc_message(snd, rcv, x, wv.reshape(4 * N_EDGES, CH))
    msg = msg2d.reshape(4, N_NODES, CH)
    m = _out_linear(msg, w_stack)
    m1 = jnp.stack([m[1], m[2], m[3]], axis=-1).reshape(N_NODES, 3 * CH)
    return jnp.concatenate([m[0], m1], axis=1)
